# Initial kernel scaffold; baseline (speedup 1.0000x reference)
import jax
import jax.numpy as jnp
from jax.experimental import pallas as pl


def kernel(pred_bboxes, pred_scores, priors, gt_labels, gt_bboxes, pad_bbox_flag):
    raise NotImplementedError("write your pallas kernel here")


# speedup vs baseline: 38.7909x; 38.7909x over previous
import jax
import jax.numpy as jnp
from jax import lax
from jax.experimental import pallas as pl

_INF = 100000000.0
_EPS = 1e-7
_NUM_CLASSES = 80
_RADIUS = 3.0
_TOPK = 13
_IOU_W = 3.0


def _assign_body(pb_ref, ps_ref, pr_ref, gl_ref, gb_ref, pf_ref,
                 lab_ref, box_ref, met_ref):
    N = pb_ref.shape[2]
    G = gb_ref.shape[1]
    f32 = jnp.float32

    pb = pb_ref[0]
    px1 = pb[0:1, :]
    py1 = pb[1:2, :]
    px2 = pb[2:3, :]
    py2 = pb[3:4, :]
    gb = gb_ref[0]
    gx1 = gb[:, 0:1]
    gy1 = gb[:, 1:2]
    gx2 = gb[:, 2:3]
    gy2 = gb[:, 3:4]
    pr = pr_ref[...]
    pcx = pr[0:1, :]
    pcy = pr[1:2, :]
    pstride = pr[2:3, :]
    padf = pf_ref[0]
    labels = gl_ref[0]

    lt_x = pcx - gx1
    lt_y = pcy - gy1
    rb_x = gx2 - pcx
    rb_y = gy2 - pcy
    min4 = jnp.minimum(jnp.minimum(lt_x, lt_y), jnp.minimum(rb_x, rb_y))
    is_in = (min4 > 0).astype(f32) * padf
    valid = jnp.sum(is_in, axis=0, keepdims=True) > 0

    gcx = (gx1 + gx2) / 2.0
    gcy = (gy1 + gy2) / 2.0
    dx = pcx - gcx
    dy = pcy - gcy
    distance = jnp.sqrt(dx * dx + dy * dy) / pstride
    distance = distance * valid.astype(f32)
    soft = jnp.exp2((distance - _RADIUS) * 3.321928094887362)

    area1 = (px2 - px1) * (py2 - py1)
    area2 = (gx2 - gx1) * (gy2 - gy1)
    ox = jnp.clip(jnp.minimum(px2, gx2) - jnp.maximum(px1, gx1), 0.0, None)
    oy = jnp.clip(jnp.minimum(py2, gy2) - jnp.maximum(py1, gy1), 0.0, None)
    overlap = ox * oy
    union = jnp.maximum(area1 + area2 - overlap, 1e-6)
    iou = overlap / union
    iou_cost = -jnp.log(iou + _EPS) * _IOU_W

    scores_t = ps_ref[0]
    C = scores_t.shape[0]
    c_iota = lax.broadcasted_iota(jnp.int32, (G, C), 1)
    onehot_l = (c_iota == labels).astype(f32)
    x = jnp.dot(onehot_l, scores_t, preferred_element_type=f32)
    sf = iou - jax.nn.sigmoid(x)
    bce = jnp.maximum(x, 0.0) - x * iou + jnp.log1p(jnp.exp(-jnp.abs(x)))
    cls_cost = bce * (sf * sf)

    cost = cls_cost + iou_cost + soft
    cost = jnp.where(valid, cost, _INF)

    n_iota = lax.broadcasted_iota(jnp.int32, (1, N), 1)

    iou_w = iou
    ksum = jnp.zeros((G, 1), f32)
    for _ in range(_TOPK):
        mv = jnp.max(iou_w, axis=1, keepdims=True)
        ksum = ksum + mv
        cand = jnp.where(iou_w == mv, n_iota, N)
        mi = jnp.min(cand, axis=1, keepdims=True)
        iou_w = jnp.where(n_iota == mi, -1.0, iou_w)
    dyn_k = jnp.clip(ksum.astype(jnp.int32), 1, None)

    gt_valid = padf > 0
    excl = jnp.zeros((G, N), jnp.bool_)
    matching = jnp.zeros((G, N), f32)
    for j in range(_TOPK):
        cmask = jnp.where(excl, jnp.inf, cost)
        mv = jnp.min(cmask, axis=1, keepdims=True)
        cand = jnp.where((cmask == mv) & (~excl), n_iota, N)
        mi = jnp.min(cand, axis=1, keepdims=True)
        hit = n_iota == mi
        excl = excl | hit
        take = hit & (j < dyn_k) & gt_valid
        matching = matching + take.astype(f32)

    cnt = jnp.sum(matching, axis=0, keepdims=True)
    g_iota = lax.broadcasted_iota(jnp.int32, (G, 1), 0)
    rmin = jnp.min(cost, axis=0, keepdims=True)
    rcand = jnp.where(cost == rmin, g_iota, G)
    rid = jnp.min(rcand, axis=0, keepdims=True)
    onehot = (g_iota == rid).astype(f32)
    final = jnp.where(cnt > 1.0, onehot, matching)

    fg = jnp.sum(final, axis=0, keepdims=True) > 0
    met = jnp.sum(final * iou, axis=0, keepdims=True)
    labf = jnp.sum(final * labels.astype(f32), axis=0, keepdims=True)
    lab = jnp.where(fg, labf.astype(jnp.int32), _NUM_CLASSES)

    bx = [jnp.sum(final * gb[:, c:c + 1], axis=0, keepdims=True)
          for c in range(4)]
    boxes = jnp.concatenate(bx, axis=0)
    boxes = jnp.where(fg, boxes, 0.0)
    met = jnp.where(fg, met, 0.0)

    lab_ref[0] = lab
    box_ref[0] = boxes
    met_ref[0] = met


def kernel(pred_bboxes, pred_scores, priors, gt_labels, gt_bboxes,
           pad_bbox_flag):
    B, N, _ = pred_bboxes.shape
    G = gt_bboxes.shape[1]
    C = pred_scores.shape[2]

    pb_t = jnp.transpose(pred_bboxes, (0, 2, 1))
    ps_t = jnp.transpose(pred_scores, (0, 2, 1))
    pr_t = jnp.transpose(priors, (1, 0))

    labs, boxes, mets = pl.pallas_call(
        _assign_body,
        grid=(B,),
        in_specs=[
            pl.BlockSpec((1, 4, N), lambda b: (b, 0, 0)),
            pl.BlockSpec((1, C, N), lambda b: (b, 0, 0)),
            pl.BlockSpec((4, N), lambda b: (0, 0)),
            pl.BlockSpec((1, G, 1), lambda b: (b, 0, 0)),
            pl.BlockSpec((1, G, 4), lambda b: (b, 0, 0)),
            pl.BlockSpec((1, G, 1), lambda b: (b, 0, 0)),
        ],
        out_specs=(
            pl.BlockSpec((1, 1, N), lambda b: (b, 0, 0)),
            pl.BlockSpec((1, 4, N), lambda b: (b, 0, 0)),
            pl.BlockSpec((1, 1, N), lambda b: (b, 0, 0)),
        ),
        out_shape=(
            jax.ShapeDtypeStruct((B, 1, N), jnp.int32),
            jax.ShapeDtypeStruct((B, 4, N), jnp.float32),
            jax.ShapeDtypeStruct((B, 1, N), jnp.float32),
        ),
    )(pb_t, ps_t, pr_t, gt_labels, gt_bboxes, pad_bbox_flag)

    return (labs.reshape(B, N),
            jnp.ones((B, N), jnp.float32),
            jnp.transpose(boxes, (0, 2, 1)),
            mets.reshape(B, N))
